# Initial kernel scaffold; baseline (speedup 1.0000x reference)
import jax
import jax.numpy as jnp
from jax.experimental import pallas as pl


def kernel(xu, xv, xe, params, adj_row, adj_col, ep_row, ep_col):
    raise NotImplementedError("write your pallas kernel here")


# speedup vs baseline: 1.0996x; 1.0996x over previous
import functools
import jax
import jax.numpy as jnp
from jax.experimental import pallas as pl
from jax.experimental.pallas import tpu as pltpu


def _mm_body(x_ref, w_ref, b_ref, o_ref, *, act):
    x = x_ref[...]
    w = w_ref[...]
    acc = jax.lax.dot_general(x, w, (((1,), (0,)), ((), ())),
                              preferred_element_type=jnp.float32)
    acc = acc + b_ref[...][None, :]
    if act == "relu":
        acc = jnp.maximum(acc, 0.0)
    elif act == "sigmoid":
        acc = jax.nn.sigmoid(acc)
    o_ref[...] = acc


@functools.partial(jax.jit, static_argnames=("act", "blk"))
def _mm(x, w, b, act="none", blk=1000):
    m, k = x.shape
    n = w.shape[1]
    pad = (-m) % blk
    if pad:
        x = jnp.concatenate([x, jnp.zeros((pad, k), x.dtype)], axis=0)
    mp = x.shape[0]
    out = pl.pallas_call(
        functools.partial(_mm_body, act=act),
        grid=(mp // blk,),
        in_specs=[
            pl.BlockSpec((blk, k), lambda i: (i, 0)),
            pl.BlockSpec((k, n), lambda i: (0, 0)),
            pl.BlockSpec((n,), lambda i: (0,)),
        ],
        out_specs=pl.BlockSpec((blk, n), lambda i: (i, 0)),
        out_shape=jax.ShapeDtypeStruct((mp, n), jnp.float32),
    )(x, w, b)
    return out[:m] if pad else out


def _mlp(h, layers, final_act="none"):
    last = len(layers) - 1
    for i, p in enumerate(layers):
        h = _mm(h, p["W"], p["b"], act=("relu" if i != last else final_act))
    return h


def _agg(feat, idx, n):
    s = jax.ops.segment_sum(feat, idx, num_segments=n)
    cnt = jax.ops.segment_sum(jnp.ones((feat.shape[0], 1), feat.dtype), idx,
                              num_segments=n)
    mean = s / jnp.maximum(cnt, 1.0)
    mx = jax.ops.segment_max(feat, idx, num_segments=n)
    mx = jnp.where(cnt > 0, mx, 0.0)
    return mean, mx


def _bean_conv(xu, xv, xe, row, col, p, self_loop, relu_out):
    nu, nv = xu.shape[0], xv.shape[0]
    mv, Mv = _agg(xv[col], row, nu)
    me_u, Me_u = _agg(xe, row, nu)
    mu, Mu = _agg(xu[row], col, nv)
    me_v, Me_v = _agg(xe, col, nv)
    u_in = ([xu] if self_loop else []) + [mv, Mv, me_u, Me_u]
    v_in = ([xv] if self_loop else []) + [mu, Mu, me_v, Me_v]
    xu_n = _mm(jnp.concatenate(u_in, axis=1), p["u"]["W"], p["u"]["b"])
    xv_n = _mm(jnp.concatenate(v_in, axis=1), p["v"]["W"], p["v"]["b"])
    if "e" in p:
        ou = p["u"]["W"].shape[1]
        we = p["e"]["W"]
        pu = _mm(xu_n, we[:ou], p["e"]["b"])
        pv = _mm(xv_n, we[ou:], jnp.zeros_like(p["e"]["b"]))
        xe_n = pu[row] + pv[col]
    else:
        xe_n = jnp.concatenate([xu_n[row], xv_n[col]], axis=1)
    if relu_out:
        xu_n = jnp.maximum(xu_n, 0.0)
        xv_n = jnp.maximum(xv_n, 0.0)
        xe_n = jnp.maximum(xe_n, 0.0)
    return xu_n, xv_n, xe_n


def kernel(xu, xv, xe, params, adj_row, adj_col, ep_row, ep_col):
    n_enc = len(params["enc"])
    for i, p in enumerate(params["enc"]):
        xu, xv, xe = _bean_conv(xu, xv, xe, adj_row, adj_col, p,
                                i != n_enc - 1, i != n_enc - 1)
    zu, zv = xu, xv
    n_dec = len(params["dec"])
    for i, p in enumerate(params["dec"]):
        xu, xv, xe = _bean_conv(xu, xv, xe, adj_row, adj_col, p,
                                True, i != n_dec - 1)
    nprob_u = _mlp(zu, params["clf"], final_act="sigmoid")
    nprob_v = _mlp(zv, params["clf"], final_act="sigmoid")
    zu2 = _mlp(zu, params["umlp"])
    zv2 = _mlp(zv, params["vmlp"])
    eprob = jax.nn.sigmoid(jnp.sum(zu2[ep_row] * zv2[ep_col], axis=1))
    return (xu, xv, xe, zu, zv, eprob, nprob_u, nprob_v)


# speedup vs baseline: 2.0668x; 1.8796x over previous
import functools
import jax
import jax.numpy as jnp
from jax import lax
from jax.experimental import pallas as pl
from jax.experimental.pallas import tpu as pltpu
from jax.experimental.pallas import tpu_sc as plsc

N_NODES = 10000
N_EDGES = 320000
NW = 32
NPW = 320
SG = 128

_SC_PARAMS = pltpu.CompilerParams(needs_layout_passes=False,
                                  use_tc_tiling_on_sc=False)


def _sread(ref, i):
    v = plsc.load_gather(ref, [jnp.broadcast_to(i, (16,)).astype(jnp.int32)])
    return v[0]


@functools.lru_cache(maxsize=None)
def _make_agg2(d1, d2, ch):
    dca, dcb = d1 // 16, d2 // 16
    dc = dca + dcb
    od = 2 * (d1 + d2)
    nsub = ch // SG
    mesh = plsc.VectorSubcoreMesh(core_axis_name="c", subcore_axis_name="s")

    @functools.partial(
        pl.kernel, mesh=mesh,
        out_type=jax.ShapeDtypeStruct((NW * NPW, od), jnp.float32),
        scratch_types=[
            pltpu.VMEM((NPW + 8,), jnp.int32),
            pltpu.VMEM((ch,), jnp.int32),
            pltpu.VMEM((ch,), jnp.int32),
            pltpu.VMEM((ch,), jnp.int32),
            pltpu.VMEM((ch,), jnp.int32),
            pltpu.VMEM((ch, d1), jnp.float32),
            pltpu.VMEM((ch, d1), jnp.float32),
            pltpu.VMEM((ch, d2), jnp.float32),
            pltpu.VMEM((ch, d2), jnp.float32),
            pltpu.VMEM((8, od), jnp.float32),
            pltpu.SemaphoreType.DMA,
            pltpu.SemaphoreType.DMA,
        ],
        compiler_params=_SC_PARAMS,
    )
    def k(ta_hbm, tb_hbm, ga_hbm, gb_hbm, st_hbm, out_hbm,
          st_v, ia0, ia1, ib0, ib1, fa0, fa1, fb0, fb1, ring_v, sem0, sem1):
        ia = (ia0, ia1)
        ib = (ib0, ib1)
        fa = (fa0, fa1)
        fb = (fb0, fb1)
        sems = (sem0, sem1)
        wid = lax.axis_index("s") * 2 + lax.axis_index("c")
        n0 = pl.multiple_of(wid * NPW, 8)
        count = jnp.minimum(NPW, N_NODES - n0)
        pltpu.sync_copy(st_hbm.at[pl.ds(n0, NPW + 8)], st_v)
        e0 = _sread(st_v, 0)
        e1 = _sread(st_v, count)
        e0a = jnp.bitwise_and(e0, -8)
        nch = (e1 - e0a + (ch - 1)) // ch

        def fire(c, b):
            off = pl.multiple_of(e0a + c * ch, 8)
            pltpu.sync_copy(ga_hbm.at[pl.ds(off, ch)], ia[b])
            pltpu.sync_copy(gb_hbm.at[pl.ds(off, ch)], ib[b])
            for j in range(nsub):
                s = pl.ds(j * SG, SG)
                pltpu.async_copy(ta_hbm.at[ia[b].at[s]], fa[b].at[s], sems[b])
                pltpu.async_copy(tb_hbm.at[ib[b].at[s]], fb[b].at[s], sems[b])

        def wait(b):
            for j in range(nsub):
                s = pl.ds(j * SG, SG)
                pltpu.make_async_copy(ta_hbm.at[ia[b].at[s]], fa[b].at[s], sems[b]).wait()
                pltpu.make_async_copy(tb_hbm.at[ib[b].at[s]], fb[b].at[s], sems[b]).wait()

        zero_acc = tuple(jnp.zeros((16,), jnp.float32) for _ in range(dc))
        ninf_acc = tuple(jnp.full((16,), -jnp.inf, jnp.float32) for _ in range(dc))

        def finalize(nrel, cur_len, acc_s, acc_m):
            slot = jnp.bitwise_and(nrel, 7)
            has = jnp.broadcast_to(cur_len > 0, (16,))
            lenf = jnp.maximum(jnp.broadcast_to(cur_len, (16,)).astype(jnp.float32), 1.0)
            row = ring_v.at[slot]
            for q in range(dca):
                row[pl.ds(q * 16, 16)] = acc_s[q] / lenf
                row[pl.ds(d1 + q * 16, 16)] = jnp.where(has, acc_m[q], 0.0)
            for q in range(dcb):
                row[pl.ds(2 * d1 + q * 16, 16)] = acc_s[dca + q] / lenf
                row[pl.ds(2 * d1 + d2 + q * 16, 16)] = jnp.where(has, acc_m[dca + q], 0.0)

            @pl.when(slot == 7)
            def _flush():
                pltpu.sync_copy(ring_v, out_hbm.at[pl.ds(n0 + nrel - 7, 8)])

        def process_chunk(state, c, b):
            base = e0a + c * ch
            pos0 = jnp.maximum(e0 - base, 0)
            hi = jnp.clip(e1 - base, 0, ch)

            def wcond(s):
                return s[0] < hi

            def wbody(s):
                pos, nrel, rem, cur_len, acc_s, acc_m = s
                t = jnp.minimum(rem, hi - pos)

                def abody(j, a):
                    s_, m_ = a
                    rowa = fa[b].at[pos + j]
                    rowb = fb[b].at[pos + j]
                    new_s, new_m = [], []
                    for q in range(dca):
                        v = rowa[pl.ds(q * 16, 16)]
                        new_s.append(s_[q] + v)
                        new_m.append(jnp.maximum(m_[q], v))
                    for q in range(dcb):
                        v = rowb[pl.ds(q * 16, 16)]
                        new_s.append(s_[dca + q] + v)
                        new_m.append(jnp.maximum(m_[dca + q], v))
                    return tuple(new_s), tuple(new_m)

                acc_s, acc_m = lax.fori_loop(0, t, abody, (acc_s, acc_m))
                pos = pos + t
                rem = rem - t
                done = rem == 0

                @pl.when(done)
                def _fin():
                    finalize(nrel, cur_len, acc_s, acc_m)

                nrel2 = jnp.where(done, nrel + 1, nrel)
                nxt = _sread(st_v, nrel2 + 1) - _sread(st_v, nrel2)
                rem2 = jnp.where(done, nxt, rem)
                cur_len2 = jnp.where(done, nxt, cur_len)
                d16 = jnp.broadcast_to(done, (16,))
                acc_s = tuple(jnp.where(d16, 0.0, x) for x in acc_s)
                acc_m = tuple(jnp.where(d16, -jnp.inf, x) for x in acc_m)
                return pos, nrel2, rem2, cur_len2, acc_s, acc_m

            nrel, rem, cur_len, acc_s, acc_m = state
            out = lax.while_loop(wcond, wbody, (pos0, nrel, rem, cur_len, acc_s, acc_m))
            return out[1:]

        @pl.when(nch > 0)
        def _prime():
            fire(0, 0)

        len0 = _sread(st_v, 1) - e0
        state0 = (jnp.int32(0), len0, len0, zero_acc, ninf_acc)

        def cbody(c2, state):
            for b in (0, 1):
                c = c2 * 2 + b

                @pl.when(c + 1 < nch)
                def _fire_next():
                    fire(c + 1, 1 - b)

                @pl.when(c < nch)
                def _wait_cur():
                    wait(b)

                state = process_chunk(state, c, b)
            return state

        state = lax.fori_loop(0, (nch + 1) // 2, cbody, state0)

        def dcond(s):
            return s[0] < count

        def dbody(s):
            nrel, rem, cur_len, acc_s, acc_m = s
            finalize(nrel, cur_len, acc_s, acc_m)
            return (nrel + 1, jnp.int32(0), jnp.int32(0), zero_acc, ninf_acc)

        lax.while_loop(dcond, dbody, state)

    return k


def _sc_aggregate(table_a, table_b, gidx_a, gidx_b, starts):
    d1, d2 = table_a.shape[1], table_b.shape[1]
    tot = d1 + d2
    ch = 512 if tot <= 64 else 256
    pad = NW * NPW + 8 - starts.shape[0]
    starts_p = jnp.concatenate([starts, jnp.full((pad,), N_EDGES, jnp.int32)])
    epad = (-gidx_a.shape[0]) % ch + ch + 8
    ga = jnp.concatenate([gidx_a, jnp.zeros((epad,), jnp.int32)])
    gb = jnp.concatenate([gidx_b, jnp.zeros((epad,), jnp.int32)])
    out = _make_agg2(d1, d2, ch)(table_a, table_b, ga, gb, starts_p)
    return out[:N_NODES]


def _mm_body(x_ref, w_ref, b_ref, o_ref, *, act):
    x = x_ref[...]
    w = w_ref[...]
    acc = jax.lax.dot_general(x, w, (((1,), (0,)), ((), ())),
                              preferred_element_type=jnp.float32)
    acc = acc + b_ref[...][None, :]
    if act == "relu":
        acc = jnp.maximum(acc, 0.0)
    elif act == "sigmoid":
        acc = jax.nn.sigmoid(acc)
    o_ref[...] = acc


@functools.partial(jax.jit, static_argnames=("act", "blk"))
def _mm(x, w, b, act="none", blk=1000):
    m, k = x.shape
    n = w.shape[1]
    pad = (-m) % blk
    if pad:
        x = jnp.concatenate([x, jnp.zeros((pad, k), x.dtype)], axis=0)
    mp = x.shape[0]
    out = pl.pallas_call(
        functools.partial(_mm_body, act=act),
        grid=(mp // blk,),
        in_specs=[
            pl.BlockSpec((blk, k), lambda i: (i, 0)),
            pl.BlockSpec((k, n), lambda i: (0, 0)),
            pl.BlockSpec((n,), lambda i: (0,)),
        ],
        out_specs=pl.BlockSpec((blk, n), lambda i: (i, 0)),
        out_shape=jax.ShapeDtypeStruct((mp, n), jnp.float32),
    )(x, w, b)
    return out[:m] if pad else out


def _mm2_body(x1_ref, w1_ref, x2_ref, w2_ref, b_ref, o_ref):
    acc = jax.lax.dot_general(x1_ref[...], w1_ref[...], (((1,), (0,)), ((), ())),
                              preferred_element_type=jnp.float32)
    acc += jax.lax.dot_general(x2_ref[...], w2_ref[...], (((1,), (0,)), ((), ())),
                               preferred_element_type=jnp.float32)
    o_ref[...] = acc + b_ref[...][None, :]


@functools.partial(jax.jit, static_argnames=("blk",))
def _mm2(x1, w1, x2, w2, b, blk=1000):
    m, k1 = x1.shape
    k2 = x2.shape[1]
    n = w1.shape[1]
    pad = (-m) % blk
    if pad:
        x1 = jnp.concatenate([x1, jnp.zeros((pad, k1), x1.dtype)], axis=0)
        x2 = jnp.concatenate([x2, jnp.zeros((pad, k2), x2.dtype)], axis=0)
    mp = x1.shape[0]
    out = pl.pallas_call(
        _mm2_body,
        grid=(mp // blk,),
        in_specs=[
            pl.BlockSpec((blk, k1), lambda i: (i, 0)),
            pl.BlockSpec((k1, n), lambda i: (0, 0)),
            pl.BlockSpec((blk, k2), lambda i: (i, 0)),
            pl.BlockSpec((k2, n), lambda i: (0, 0)),
            pl.BlockSpec((n,), lambda i: (0,)),
        ],
        out_specs=pl.BlockSpec((blk, n), lambda i: (i, 0)),
        out_shape=jax.ShapeDtypeStruct((mp, n), jnp.float32),
    )(x1, w1, x2, w2, b)
    return out[:m] if pad else out


def _mlp(h, layers, final_act="none"):
    last = len(layers) - 1
    for i, p in enumerate(layers):
        h = _mm(h, p["W"], p["b"], act=("relu" if i != last else final_act))
    return h


def _bean_conv(xu, xv, xe, row, col, srt, p, self_loop, relu_out):
    perm_r, starts_r, nbr_r, perm_c, starts_c, nbr_c = srt
    agg_u = _sc_aggregate(xv, xe, nbr_r, perm_r, starts_r)
    agg_v = _sc_aggregate(xu, xe, nbr_c, perm_c, starts_c)
    if self_loop:
        du = xu.shape[1]
        dv = xv.shape[1]
        xu_n = _mm2(xu, p["u"]["W"][:du], agg_u, p["u"]["W"][du:], p["u"]["b"])
        xv_n = _mm2(xv, p["v"]["W"][:dv], agg_v, p["v"]["W"][dv:], p["v"]["b"])
    else:
        xu_n = _mm(agg_u, p["u"]["W"], p["u"]["b"])
        xv_n = _mm(agg_v, p["v"]["W"], p["v"]["b"])
    if "e" in p:
        ou = p["u"]["W"].shape[1]
        we = p["e"]["W"]
        pu = _mm(xu_n, we[:ou], p["e"]["b"])
        pv = _mm(xv_n, we[ou:], jnp.zeros_like(p["e"]["b"]))
        xe_n = pu[row] + pv[col]
    else:
        xe_n = jnp.concatenate([xu_n[row], xv_n[col]], axis=1)
    if relu_out:
        xu_n = jnp.maximum(xu_n, 0.0)
        xv_n = jnp.maximum(xv_n, 0.0)
        xe_n = jnp.maximum(xe_n, 0.0)
    return xu_n, xv_n, xe_n


def kernel(xu, xv, xe, params, adj_row, adj_col, ep_row, ep_col):
    perm_r = jnp.argsort(adj_row).astype(jnp.int32)
    perm_c = jnp.argsort(adj_col).astype(jnp.int32)
    row_s = adj_row[perm_r]
    col_s = adj_col[perm_c]
    grid = jnp.arange(N_NODES + 1, dtype=jnp.int32)
    starts_r = jnp.searchsorted(row_s, grid).astype(jnp.int32)
    starts_c = jnp.searchsorted(col_s, grid).astype(jnp.int32)
    nbr_r = adj_col[perm_r]
    nbr_c = adj_row[perm_c]
    srt = (perm_r, starts_r, nbr_r, perm_c, starts_c, nbr_c)

    n_enc = len(params["enc"])
    for i, p in enumerate(params["enc"]):
        xu, xv, xe = _bean_conv(xu, xv, xe, adj_row, adj_col, srt, p,
                                i != n_enc - 1, i != n_enc - 1)
    zu, zv = xu, xv
    n_dec = len(params["dec"])
    for i, p in enumerate(params["dec"]):
        xu, xv, xe = _bean_conv(xu, xv, xe, adj_row, adj_col, srt, p,
                                True, i != n_dec - 1)
    nprob_u = _mlp(zu, params["clf"], final_act="sigmoid")
    nprob_v = _mlp(zv, params["clf"], final_act="sigmoid")
    zu2 = _mlp(zu, params["umlp"])
    zv2 = _mlp(zv, params["vmlp"])
    eprob = jax.nn.sigmoid(jnp.sum(zu2[ep_row] * zv2[ep_col], axis=1))
    return (xu, xv, xe, zu, zv, eprob, nprob_u, nprob_v)


# speedup vs baseline: 2.9542x; 1.4294x over previous
import functools
import jax
import jax.numpy as jnp
from jax import lax
from jax.experimental import pallas as pl
from jax.experimental.pallas import tpu as pltpu
from jax.experimental.pallas import tpu_sc as plsc

N_NODES = 10000
N_EDGES = 320000
NW = 32
NPW = 320
SG = 128

_SC_PARAMS = pltpu.CompilerParams(needs_layout_passes=False,
                                  use_tc_tiling_on_sc=False)


def _sread(ref, i):
    v = plsc.load_gather(ref, [jnp.broadcast_to(i, (16,)).astype(jnp.int32)])
    return v[0]


@functools.lru_cache(maxsize=None)
def _make_agg2(d1, d2, ch):
    dca, dcb = d1 // 16, d2 // 16
    dc = dca + dcb
    od = 2 * (d1 + d2)
    nsub = ch // SG
    mesh = plsc.VectorSubcoreMesh(core_axis_name="c", subcore_axis_name="s")

    @functools.partial(
        pl.kernel, mesh=mesh,
        out_type=jax.ShapeDtypeStruct((NW * NPW, od), jnp.float32),
        scratch_types=[
            pltpu.VMEM((NPW + 8,), jnp.int32),
            pltpu.VMEM((ch,), jnp.int32),
            pltpu.VMEM((ch,), jnp.int32),
            pltpu.VMEM((ch,), jnp.int32),
            pltpu.VMEM((ch,), jnp.int32),
            pltpu.VMEM((ch, d1), jnp.float32),
            pltpu.VMEM((ch, d1), jnp.float32),
            pltpu.VMEM((ch, d2), jnp.float32),
            pltpu.VMEM((ch, d2), jnp.float32),
            pltpu.VMEM((8, od), jnp.float32),
            pltpu.SemaphoreType.DMA,
            pltpu.SemaphoreType.DMA,
        ],
        compiler_params=_SC_PARAMS,
    )
    def k(ta_hbm, tb_hbm, ga_hbm, gb_hbm, st_hbm, out_hbm,
          st_v, ia0, ia1, ib0, ib1, fa0, fa1, fb0, fb1, ring_v, sem0, sem1):
        ia = (ia0, ia1)
        ib = (ib0, ib1)
        fa = (fa0, fa1)
        fb = (fb0, fb1)
        sems = (sem0, sem1)
        wid = lax.axis_index("s") * 2 + lax.axis_index("c")
        n0 = pl.multiple_of(wid * NPW, 8)
        count = jnp.minimum(NPW, N_NODES - n0)
        pltpu.sync_copy(st_hbm.at[pl.ds(n0, NPW + 8)], st_v)
        e0 = _sread(st_v, 0)
        e1 = _sread(st_v, count)
        e0a = jnp.bitwise_and(e0, -8)
        nch = (e1 - e0a + (ch - 1)) // ch

        def fire(c, b):
            off = pl.multiple_of(e0a + c * ch, 8)
            pltpu.sync_copy(ga_hbm.at[pl.ds(off, ch)], ia[b])
            pltpu.sync_copy(gb_hbm.at[pl.ds(off, ch)], ib[b])
            for j in range(nsub):
                s = pl.ds(j * SG, SG)
                pltpu.async_copy(ta_hbm.at[ia[b].at[s]], fa[b].at[s], sems[b])
                pltpu.async_copy(tb_hbm.at[ib[b].at[s]], fb[b].at[s], sems[b])

        def wait(b):
            for j in range(nsub):
                s = pl.ds(j * SG, SG)
                pltpu.make_async_copy(ta_hbm.at[ia[b].at[s]], fa[b].at[s], sems[b]).wait()
                pltpu.make_async_copy(tb_hbm.at[ib[b].at[s]], fb[b].at[s], sems[b]).wait()

        zero_acc = tuple(jnp.zeros((16,), jnp.float32) for _ in range(dc))
        ninf_acc = tuple(jnp.full((16,), -jnp.inf, jnp.float32) for _ in range(dc))

        def finalize(nrel, cur_len, acc_s, acc_m):
            slot = jnp.bitwise_and(nrel, 7)
            has = jnp.broadcast_to(cur_len > 0, (16,))
            lenf = jnp.maximum(jnp.broadcast_to(cur_len, (16,)).astype(jnp.float32), 1.0)
            row = ring_v.at[slot]
            for q in range(dca):
                row[pl.ds(q * 16, 16)] = acc_s[q] / lenf
                row[pl.ds(d1 + q * 16, 16)] = jnp.where(has, acc_m[q], 0.0)
            for q in range(dcb):
                row[pl.ds(2 * d1 + q * 16, 16)] = acc_s[dca + q] / lenf
                row[pl.ds(2 * d1 + d2 + q * 16, 16)] = jnp.where(has, acc_m[dca + q], 0.0)

            @pl.when(slot == 7)
            def _flush():
                pltpu.sync_copy(ring_v, out_hbm.at[pl.ds(n0 + nrel - 7, 8)])

        def process_chunk(state, c, b):
            base = e0a + c * ch
            pos0 = jnp.maximum(e0 - base, 0)
            hi = jnp.clip(e1 - base, 0, ch)

            def wcond(s):
                return s[0] < hi

            def wbody(s):
                pos, nrel, rem, cur_len, acc_s, acc_m = s
                t = jnp.minimum(rem, hi - pos)

                def abody(j, a):
                    s_, m_ = a
                    rowa = fa[b].at[pos + j]
                    rowb = fb[b].at[pos + j]
                    new_s, new_m = [], []
                    for q in range(dca):
                        v = rowa[pl.ds(q * 16, 16)]
                        new_s.append(s_[q] + v)
                        new_m.append(jnp.maximum(m_[q], v))
                    for q in range(dcb):
                        v = rowb[pl.ds(q * 16, 16)]
                        new_s.append(s_[dca + q] + v)
                        new_m.append(jnp.maximum(m_[dca + q], v))
                    return tuple(new_s), tuple(new_m)

                acc_s, acc_m = lax.fori_loop(0, t, abody, (acc_s, acc_m))
                pos = pos + t
                rem = rem - t
                done = rem == 0

                @pl.when(done)
                def _fin():
                    finalize(nrel, cur_len, acc_s, acc_m)

                nrel2 = jnp.where(done, nrel + 1, nrel)
                nxt = _sread(st_v, nrel2 + 1) - _sread(st_v, nrel2)
                rem2 = jnp.where(done, nxt, rem)
                cur_len2 = jnp.where(done, nxt, cur_len)
                d16 = jnp.broadcast_to(done, (16,))
                acc_s = tuple(jnp.where(d16, 0.0, x) for x in acc_s)
                acc_m = tuple(jnp.where(d16, -jnp.inf, x) for x in acc_m)
                return pos, nrel2, rem2, cur_len2, acc_s, acc_m

            nrel, rem, cur_len, acc_s, acc_m = state
            out = lax.while_loop(wcond, wbody, (pos0, nrel, rem, cur_len, acc_s, acc_m))
            return out[1:]

        @pl.when(nch > 0)
        def _prime():
            fire(0, 0)

        len0 = _sread(st_v, 1) - e0
        state0 = (jnp.int32(0), len0, len0, zero_acc, ninf_acc)

        def cbody(c2, state):
            for b in (0, 1):
                c = c2 * 2 + b

                @pl.when(c + 1 < nch)
                def _fire_next():
                    fire(c + 1, 1 - b)

                @pl.when(c < nch)
                def _wait_cur():
                    wait(b)

                state = process_chunk(state, c, b)
            return state

        state = lax.fori_loop(0, (nch + 1) // 2, cbody, state0)

        def dcond(s):
            return s[0] < count

        def dbody(s):
            nrel, rem, cur_len, acc_s, acc_m = s
            finalize(nrel, cur_len, acc_s, acc_m)
            return (nrel + 1, jnp.int32(0), jnp.int32(0), zero_acc, ninf_acc)

        lax.while_loop(dcond, dbody, state)

    return k


@functools.lru_cache(maxsize=None)
def _make_edge_map(da, db, mode, ch, epw):
    dout = da + db if mode == "concat" else da
    nsub = ch // SG
    nch = epw // ch
    assert nch % 2 == 0
    mesh = plsc.VectorSubcoreMesh(core_axis_name="c", subcore_axis_name="s")
    scratch = [
        pltpu.VMEM((ch,), jnp.int32),
        pltpu.VMEM((ch,), jnp.int32),
        pltpu.VMEM((ch,), jnp.int32),
        pltpu.VMEM((ch,), jnp.int32),
        pltpu.VMEM((ch, da), jnp.float32),
        pltpu.VMEM((ch, da), jnp.float32),
        pltpu.VMEM((ch, db), jnp.float32),
        pltpu.VMEM((ch, db), jnp.float32),
        pltpu.SemaphoreType.DMA,
        pltpu.SemaphoreType.DMA,
    ]
    if mode != "concat":
        scratch += [pltpu.VMEM((ch, dout), jnp.float32),
                    pltpu.VMEM((ch, dout), jnp.float32)]

    @functools.partial(
        pl.kernel, mesh=mesh,
        out_type=jax.ShapeDtypeStruct((NW * epw, dout), jnp.float32),
        scratch_types=scratch,
        compiler_params=_SC_PARAMS,
    )
    def k(ta_hbm, tb_hbm, ia_hbm, ib_hbm, out_hbm,
          ia0, ia1, ib0, ib1, fa0, fa1, fb0, fb1, sem0, sem1, *obufs):
        ia = (ia0, ia1)
        ib = (ib0, ib1)
        fa = (fa0, fa1)
        fb = (fb0, fb1)
        sems = (sem0, sem1)
        wid = lax.axis_index("s") * 2 + lax.axis_index("c")
        base_e = pl.multiple_of(wid * epw, 8)

        def fire(c, b):
            off = pl.multiple_of(base_e + c * ch, 8)
            pltpu.sync_copy(ia_hbm.at[pl.ds(off, ch)], ia[b])
            pltpu.sync_copy(ib_hbm.at[pl.ds(off, ch)], ib[b])
            for j in range(nsub):
                s = pl.ds(j * SG, SG)
                pltpu.async_copy(ta_hbm.at[ia[b].at[s]], fa[b].at[s], sems[b])
                pltpu.async_copy(tb_hbm.at[ib[b].at[s]], fb[b].at[s], sems[b])

        def wait(b):
            for j in range(nsub):
                s = pl.ds(j * SG, SG)
                pltpu.make_async_copy(ta_hbm.at[ia[b].at[s]], fa[b].at[s], sems[b]).wait()
                pltpu.make_async_copy(tb_hbm.at[ib[b].at[s]], fb[b].at[s], sems[b]).wait()

        def compute(c, b):
            rows = pl.ds(base_e + c * ch, ch)
            if mode == "concat":
                pltpu.sync_copy(fa[b], out_hbm.at[rows, pl.ds(0, da)])
                pltpu.sync_copy(fb[b], out_hbm.at[rows, pl.ds(da, db)])
                return
            ob = obufs[b]

            def rbody(r, _):
                rowa = fa[b].at[r]
                rowb = fb[b].at[r]
                rowo = ob.at[r]
                for q in range(da // 16):
                    s = pl.ds(q * 16, 16)
                    va = rowa[s]
                    vb = rowb[s]
                    if mode == "mul":
                        o = va * vb
                    else:
                        o = va + vb
                        if mode == "add_relu":
                            o = jnp.maximum(o, 0.0)
                    rowo[s] = o
                return 0

            lax.fori_loop(0, ch, rbody, 0)
            pltpu.sync_copy(ob, out_hbm.at[rows])

        fire(0, 0)

        def cbody(c2, carry):
            for b in (0, 1):
                c = c2 * 2 + b

                @pl.when(c + 1 < nch)
                def _fire_next():
                    fire(c + 1, 1 - b)

                wait(b)
                compute(c, b)
            return carry

        lax.fori_loop(0, nch // 2, cbody, 0)

    return k


def _sc_edge_map(ta, tb, ia, ib, mode):
    da, db = ta.shape[1], tb.shape[1]
    e = ia.shape[0]
    ch = 512 if da + db <= 64 else 256
    epw = -(-e // (NW * ch)) * ch
    ep = NW * epw
    if ep != e:
        ia = jnp.concatenate([ia, jnp.zeros((ep - e,), jnp.int32)])
        ib = jnp.concatenate([ib, jnp.zeros((ep - e,), jnp.int32)])
    out = _make_edge_map(da, db, mode, ch, epw)(ta, tb, ia, ib)
    return out[:e]


def _rowsum_sigmoid_body(x_ref, o_ref):
    o_ref[...] = jax.nn.sigmoid(jnp.sum(x_ref[...], axis=1))


@jax.jit
def _rowsum_sigmoid(x):
    morig, k = x.shape
    blk = 8192
    pad = (-morig) % blk
    if pad:
        x = jnp.concatenate([x, jnp.zeros((pad, k), x.dtype)], axis=0)
    m = x.shape[0]
    out = _rowsum_sigmoid_call(x, blk)
    return out[:morig]


@functools.partial(jax.jit, static_argnames=("blk",))
def _rowsum_sigmoid_call(x, blk):
    m, k = x.shape
    return pl.pallas_call(
        _rowsum_sigmoid_body,
        grid=(m // blk,),
        in_specs=[pl.BlockSpec((blk, k), lambda i: (i, 0))],
        out_specs=pl.BlockSpec((blk,), lambda i: (i,)),
        out_shape=jax.ShapeDtypeStruct((m,), jnp.float32),
    )(x)


def _sc_aggregate(table_a, table_b, gidx_a, gidx_b, starts):
    d1, d2 = table_a.shape[1], table_b.shape[1]
    tot = d1 + d2
    ch = 512 if tot <= 64 else 256
    pad = NW * NPW + 8 - starts.shape[0]
    starts_p = jnp.concatenate([starts, jnp.full((pad,), N_EDGES, jnp.int32)])
    epad = (-gidx_a.shape[0]) % ch + ch + 8
    ga = jnp.concatenate([gidx_a, jnp.zeros((epad,), jnp.int32)])
    gb = jnp.concatenate([gidx_b, jnp.zeros((epad,), jnp.int32)])
    out = _make_agg2(d1, d2, ch)(table_a, table_b, ga, gb, starts_p)
    return out[:N_NODES]


def _mm_body(x_ref, w_ref, b_ref, o_ref, *, act):
    x = x_ref[...]
    w = w_ref[...]
    acc = jax.lax.dot_general(x, w, (((1,), (0,)), ((), ())),
                              preferred_element_type=jnp.float32)
    acc = acc + b_ref[...][None, :]
    if act == "relu":
        acc = jnp.maximum(acc, 0.0)
    elif act == "sigmoid":
        acc = jax.nn.sigmoid(acc)
    o_ref[...] = acc


@functools.partial(jax.jit, static_argnames=("act", "blk"))
def _mm(x, w, b, act="none", blk=1000):
    m, k = x.shape
    n = w.shape[1]
    pad = (-m) % blk
    if pad:
        x = jnp.concatenate([x, jnp.zeros((pad, k), x.dtype)], axis=0)
    mp = x.shape[0]
    out = pl.pallas_call(
        functools.partial(_mm_body, act=act),
        grid=(mp // blk,),
        in_specs=[
            pl.BlockSpec((blk, k), lambda i: (i, 0)),
            pl.BlockSpec((k, n), lambda i: (0, 0)),
            pl.BlockSpec((n,), lambda i: (0,)),
        ],
        out_specs=pl.BlockSpec((blk, n), lambda i: (i, 0)),
        out_shape=jax.ShapeDtypeStruct((mp, n), jnp.float32),
    )(x, w, b)
    return out[:m] if pad else out


def _mm2_body(x1_ref, w1_ref, x2_ref, w2_ref, b_ref, o_ref):
    acc = jax.lax.dot_general(x1_ref[...], w1_ref[...], (((1,), (0,)), ((), ())),
                              preferred_element_type=jnp.float32)
    acc += jax.lax.dot_general(x2_ref[...], w2_ref[...], (((1,), (0,)), ((), ())),
                               preferred_element_type=jnp.float32)
    o_ref[...] = acc + b_ref[...][None, :]


@functools.partial(jax.jit, static_argnames=("blk",))
def _mm2(x1, w1, x2, w2, b, blk=1000):
    m, k1 = x1.shape
    k2 = x2.shape[1]
    n = w1.shape[1]
    pad = (-m) % blk
    if pad:
        x1 = jnp.concatenate([x1, jnp.zeros((pad, k1), x1.dtype)], axis=0)
        x2 = jnp.concatenate([x2, jnp.zeros((pad, k2), x2.dtype)], axis=0)
    mp = x1.shape[0]
    out = pl.pallas_call(
        _mm2_body,
        grid=(mp // blk,),
        in_specs=[
            pl.BlockSpec((blk, k1), lambda i: (i, 0)),
            pl.BlockSpec((k1, n), lambda i: (0, 0)),
            pl.BlockSpec((blk, k2), lambda i: (i, 0)),
            pl.BlockSpec((k2, n), lambda i: (0, 0)),
            pl.BlockSpec((n,), lambda i: (0,)),
        ],
        out_specs=pl.BlockSpec((blk, n), lambda i: (i, 0)),
        out_shape=jax.ShapeDtypeStruct((mp, n), jnp.float32),
    )(x1, w1, x2, w2, b)
    return out[:m] if pad else out


def _mlp(h, layers, final_act="none"):
    last = len(layers) - 1
    for i, p in enumerate(layers):
        h = _mm(h, p["W"], p["b"], act=("relu" if i != last else final_act))
    return h


def _bean_conv(xu, xv, xe, row, col, srt, p, self_loop, relu_out):
    perm_r, starts_r, nbr_r, perm_c, starts_c, nbr_c = srt
    agg_u = _sc_aggregate(xv, xe, nbr_r, perm_r, starts_r)
    agg_v = _sc_aggregate(xu, xe, nbr_c, perm_c, starts_c)
    if self_loop:
        du = xu.shape[1]
        dv = xv.shape[1]
        xu_n = _mm2(xu, p["u"]["W"][:du], agg_u, p["u"]["W"][du:], p["u"]["b"])
        xv_n = _mm2(xv, p["v"]["W"][:dv], agg_v, p["v"]["W"][dv:], p["v"]["b"])
    else:
        xu_n = _mm(agg_u, p["u"]["W"], p["u"]["b"])
        xv_n = _mm(agg_v, p["v"]["W"], p["v"]["b"])
    if "e" in p:
        ou = p["u"]["W"].shape[1]
        we = p["e"]["W"]
        pu = _mm(xu_n, we[:ou], p["e"]["b"])
        pv = _mm(xv_n, we[ou:], jnp.zeros_like(p["e"]["b"]))
        xe_n = _sc_edge_map(pu, pv, row, col, "add_relu" if relu_out else "add")
    else:
        xe_n = _sc_edge_map(xu_n, xv_n, row, col, "concat")
    if relu_out:
        xu_n = jnp.maximum(xu_n, 0.0)
        xv_n = jnp.maximum(xv_n, 0.0)
    return xu_n, xv_n, xe_n


def kernel(xu, xv, xe, params, adj_row, adj_col, ep_row, ep_col):
    perm_r = jnp.argsort(adj_row).astype(jnp.int32)
    perm_c = jnp.argsort(adj_col).astype(jnp.int32)
    row_s = adj_row[perm_r]
    col_s = adj_col[perm_c]
    grid = jnp.arange(N_NODES + 1, dtype=jnp.int32)
    starts_r = jnp.searchsorted(row_s, grid).astype(jnp.int32)
    starts_c = jnp.searchsorted(col_s, grid).astype(jnp.int32)
    nbr_r = adj_col[perm_r]
    nbr_c = adj_row[perm_c]
    srt = (perm_r, starts_r, nbr_r, perm_c, starts_c, nbr_c)

    n_enc = len(params["enc"])
    for i, p in enumerate(params["enc"]):
        xu, xv, xe = _bean_conv(xu, xv, xe, adj_row, adj_col, srt, p,
                                i != n_enc - 1, i != n_enc - 1)
    zu, zv = xu, xv
    n_dec = len(params["dec"])
    for i, p in enumerate(params["dec"]):
        xu, xv, xe = _bean_conv(xu, xv, xe, adj_row, adj_col, srt, p,
                                True, i != n_dec - 1)
    nprob_u = _mlp(zu, params["clf"], final_act="sigmoid")
    nprob_v = _mlp(zv, params["clf"], final_act="sigmoid")
    zu2 = _mlp(zu, params["umlp"])
    zv2 = _mlp(zv, params["vmlp"])
    g = _sc_edge_map(zu2, zv2, ep_row, ep_col, "mul")
    eprob = _rowsum_sigmoid(g)
    return (xu, xv, xe, zu, zv, eprob, nprob_u, nprob_v)


# speedup vs baseline: 3.1815x; 1.0770x over previous
import functools
import jax
import jax.numpy as jnp
from jax import lax
from jax.experimental import pallas as pl
from jax.experimental.pallas import tpu as pltpu
from jax.experimental.pallas import tpu_sc as plsc

N_NODES = 10000
N_EDGES = 320000
NW = 32
NPW = 320
SG = 128

_SC_PARAMS = pltpu.CompilerParams(needs_layout_passes=False,
                                  use_tc_tiling_on_sc=False)


def _sread(ref, i):
    v = plsc.load_gather(ref, [jnp.broadcast_to(i, (16,)).astype(jnp.int32)])
    return v[0]


@functools.lru_cache(maxsize=None)
def _make_agg2(d1, d2, ch, npw, n_real):
    dca, dcb = d1 // 16, d2 // 16
    dc = dca + dcb
    od = 2 * (d1 + d2)
    nsub = ch // SG
    mesh = plsc.VectorSubcoreMesh(core_axis_name="c", subcore_axis_name="s")

    @functools.partial(
        pl.kernel, mesh=mesh,
        out_type=jax.ShapeDtypeStruct((NW * npw, od), jnp.float32),
        scratch_types=[
            pltpu.VMEM((npw + 8,), jnp.int32),
            pltpu.VMEM((ch,), jnp.int32),
            pltpu.VMEM((ch,), jnp.int32),
            pltpu.VMEM((ch,), jnp.int32),
            pltpu.VMEM((ch,), jnp.int32),
            pltpu.VMEM((ch, d1), jnp.float32),
            pltpu.VMEM((ch, d1), jnp.float32),
            pltpu.VMEM((ch, d2), jnp.float32),
            pltpu.VMEM((ch, d2), jnp.float32),
            pltpu.VMEM((8, od), jnp.float32),
            pltpu.SemaphoreType.DMA,
            pltpu.SemaphoreType.DMA,
        ],
        compiler_params=_SC_PARAMS,
    )
    def k(ta_hbm, tb_hbm, ga_hbm, gb_hbm, st_hbm, out_hbm,
          st_v, ia0, ia1, ib0, ib1, fa0, fa1, fb0, fb1, ring_v, sem0, sem1):
        ia = (ia0, ia1)
        ib = (ib0, ib1)
        fa = (fa0, fa1)
        fb = (fb0, fb1)
        sems = (sem0, sem1)
        wid = lax.axis_index("s") * 2 + lax.axis_index("c")
        n0 = pl.multiple_of(wid * npw, 8)
        count = jnp.minimum(npw, n_real - n0)
        pltpu.sync_copy(st_hbm.at[pl.ds(n0, npw + 8)], st_v)
        e0 = _sread(st_v, 0)
        e1 = _sread(st_v, count)
        e0a = jnp.bitwise_and(e0, -8)
        nch = (e1 - e0a + (ch - 1)) // ch

        def fire(c, b):
            off = pl.multiple_of(e0a + c * ch, 8)
            pltpu.sync_copy(ga_hbm.at[pl.ds(off, ch)], ia[b])
            pltpu.sync_copy(gb_hbm.at[pl.ds(off, ch)], ib[b])
            for j in range(nsub):
                s = pl.ds(j * SG, SG)
                pltpu.async_copy(ta_hbm.at[ia[b].at[s]], fa[b].at[s], sems[b])
                pltpu.async_copy(tb_hbm.at[ib[b].at[s]], fb[b].at[s], sems[b])

        def wait(b):
            for j in range(nsub):
                s = pl.ds(j * SG, SG)
                pltpu.make_async_copy(ta_hbm.at[ia[b].at[s]], fa[b].at[s], sems[b]).wait()
                pltpu.make_async_copy(tb_hbm.at[ib[b].at[s]], fb[b].at[s], sems[b]).wait()

        zero_acc = tuple(jnp.zeros((16,), jnp.float32) for _ in range(dc))
        ninf_acc = tuple(jnp.full((16,), -jnp.inf, jnp.float32) for _ in range(dc))

        def finalize(nrel, cur_len, acc_s, acc_m):
            slot = jnp.bitwise_and(nrel, 7)
            has = jnp.broadcast_to(cur_len > 0, (16,))
            lenf = jnp.maximum(jnp.broadcast_to(cur_len, (16,)).astype(jnp.float32), 1.0)
            row = ring_v.at[slot]
            for q in range(dca):
                row[pl.ds(q * 16, 16)] = acc_s[q] / lenf
                row[pl.ds(d1 + q * 16, 16)] = jnp.where(has, acc_m[q], 0.0)
            for q in range(dcb):
                row[pl.ds(2 * d1 + q * 16, 16)] = acc_s[dca + q] / lenf
                row[pl.ds(2 * d1 + d2 + q * 16, 16)] = jnp.where(has, acc_m[dca + q], 0.0)

            @pl.when(slot == 7)
            def _flush():
                pltpu.sync_copy(ring_v, out_hbm.at[pl.ds(n0 + nrel - 7, 8)])

        def process_chunk(state, c, b):
            base = e0a + c * ch
            pos0 = jnp.maximum(e0 - base, 0)
            hi = jnp.clip(e1 - base, 0, ch)

            def wcond(s):
                return s[0] < hi

            def wbody(s):
                pos, nrel, rem, cur_len, acc_s, acc_m = s
                t = jnp.minimum(rem, hi - pos)

                def abody(j, a):
                    s_, m_ = a
                    rowa = fa[b].at[pos + j]
                    rowb = fb[b].at[pos + j]
                    new_s, new_m = [], []
                    for q in range(dca):
                        v = rowa[pl.ds(q * 16, 16)]
                        new_s.append(s_[q] + v)
                        new_m.append(jnp.maximum(m_[q], v))
                    for q in range(dcb):
                        v = rowb[pl.ds(q * 16, 16)]
                        new_s.append(s_[dca + q] + v)
                        new_m.append(jnp.maximum(m_[dca + q], v))
                    return tuple(new_s), tuple(new_m)

                acc_s, acc_m = lax.fori_loop(0, t, abody, (acc_s, acc_m))
                pos = pos + t
                rem = rem - t
                done = rem == 0

                @pl.when(done)
                def _fin():
                    finalize(nrel, cur_len, acc_s, acc_m)

                nrel2 = jnp.where(done, nrel + 1, nrel)
                nxt = _sread(st_v, nrel2 + 1) - _sread(st_v, nrel2)
                rem2 = jnp.where(done, nxt, rem)
                cur_len2 = jnp.where(done, nxt, cur_len)
                d16 = jnp.broadcast_to(done, (16,))
                acc_s = tuple(jnp.where(d16, 0.0, x) for x in acc_s)
                acc_m = tuple(jnp.where(d16, -jnp.inf, x) for x in acc_m)
                return pos, nrel2, rem2, cur_len2, acc_s, acc_m

            nrel, rem, cur_len, acc_s, acc_m = state
            out = lax.while_loop(wcond, wbody, (pos0, nrel, rem, cur_len, acc_s, acc_m))
            return out[1:]

        @pl.when(nch > 0)
        def _prime():
            fire(0, 0)

        len0 = _sread(st_v, 1) - e0
        state0 = (jnp.int32(0), len0, len0, zero_acc, ninf_acc)

        def cbody(c2, state):
            for b in (0, 1):
                c = c2 * 2 + b

                @pl.when(c + 1 < nch)
                def _fire_next():
                    fire(c + 1, 1 - b)

                @pl.when(c < nch)
                def _wait_cur():
                    wait(b)

                state = process_chunk(state, c, b)
            return state

        state = lax.fori_loop(0, (nch + 1) // 2, cbody, state0)

        def dcond(s):
            return s[0] < count

        def dbody(s):
            nrel, rem, cur_len, acc_s, acc_m = s
            finalize(nrel, cur_len, acc_s, acc_m)
            return (nrel + 1, jnp.int32(0), jnp.int32(0), zero_acc, ninf_acc)

        lax.while_loop(dcond, dbody, state)

    return k


@functools.lru_cache(maxsize=None)
def _make_edge_map(da, db, mode, ch, epw):
    dout = da + db if mode == "concat" else da
    nsub = ch // SG
    nch = epw // ch
    assert nch % 2 == 0
    mesh = plsc.VectorSubcoreMesh(core_axis_name="c", subcore_axis_name="s")
    scratch = [
        pltpu.VMEM((ch,), jnp.int32),
        pltpu.VMEM((ch,), jnp.int32),
        pltpu.VMEM((ch,), jnp.int32),
        pltpu.VMEM((ch,), jnp.int32),
        pltpu.VMEM((ch, da), jnp.float32),
        pltpu.VMEM((ch, da), jnp.float32),
        pltpu.VMEM((ch, db), jnp.float32),
        pltpu.VMEM((ch, db), jnp.float32),
        pltpu.SemaphoreType.DMA,
        pltpu.SemaphoreType.DMA,
    ]
    if mode != "concat":
        scratch += [pltpu.VMEM((ch, dout), jnp.float32),
                    pltpu.VMEM((ch, dout), jnp.float32)]

    @functools.partial(
        pl.kernel, mesh=mesh,
        out_type=jax.ShapeDtypeStruct((NW * epw, dout), jnp.float32),
        scratch_types=scratch,
        compiler_params=_SC_PARAMS,
    )
    def k(ta_hbm, tb_hbm, ia_hbm, ib_hbm, out_hbm,
          ia0, ia1, ib0, ib1, fa0, fa1, fb0, fb1, sem0, sem1, *obufs):
        ia = (ia0, ia1)
        ib = (ib0, ib1)
        fa = (fa0, fa1)
        fb = (fb0, fb1)
        sems = (sem0, sem1)
        wid = lax.axis_index("s") * 2 + lax.axis_index("c")
        base_e = pl.multiple_of(wid * epw, 8)

        def fire(c, b):
            off = pl.multiple_of(base_e + c * ch, 8)
            pltpu.sync_copy(ia_hbm.at[pl.ds(off, ch)], ia[b])
            pltpu.sync_copy(ib_hbm.at[pl.ds(off, ch)], ib[b])
            for j in range(nsub):
                s = pl.ds(j * SG, SG)
                pltpu.async_copy(ta_hbm.at[ia[b].at[s]], fa[b].at[s], sems[b])
                pltpu.async_copy(tb_hbm.at[ib[b].at[s]], fb[b].at[s], sems[b])

        def wait(b):
            for j in range(nsub):
                s = pl.ds(j * SG, SG)
                pltpu.make_async_copy(ta_hbm.at[ia[b].at[s]], fa[b].at[s], sems[b]).wait()
                pltpu.make_async_copy(tb_hbm.at[ib[b].at[s]], fb[b].at[s], sems[b]).wait()

        def compute(c, b):
            rows = pl.ds(base_e + c * ch, ch)
            if mode == "concat":
                pltpu.sync_copy(fa[b], out_hbm.at[rows, pl.ds(0, da)])
                pltpu.sync_copy(fb[b], out_hbm.at[rows, pl.ds(da, db)])
                return
            ob = obufs[b]

            def rbody(r, _):
                rowa = fa[b].at[r]
                rowb = fb[b].at[r]
                rowo = ob.at[r]
                for q in range(da // 16):
                    s = pl.ds(q * 16, 16)
                    va = rowa[s]
                    vb = rowb[s]
                    if mode == "mul":
                        o = va * vb
                    else:
                        o = va + vb
                        if mode == "add_relu":
                            o = jnp.maximum(o, 0.0)
                    rowo[s] = o
                return 0

            lax.fori_loop(0, ch, rbody, 0)
            pltpu.sync_copy(ob, out_hbm.at[rows])

        fire(0, 0)

        def cbody(c2, carry):
            for b in (0, 1):
                c = c2 * 2 + b

                @pl.when(c + 1 < nch)
                def _fire_next():
                    fire(c + 1, 1 - b)

                wait(b)
                compute(c, b)
            return carry

        lax.fori_loop(0, nch // 2, cbody, 0)

    return k


def _sc_edge_map(ta, tb, ia, ib, mode):
    da, db = ta.shape[1], tb.shape[1]
    e = ia.shape[0]
    ch = 512 if da + db <= 64 else 256
    epw = -(-e // (NW * ch)) * ch
    ep = NW * epw
    if ep != e:
        ia = jnp.concatenate([ia, jnp.zeros((ep - e,), jnp.int32)])
        ib = jnp.concatenate([ib, jnp.zeros((ep - e,), jnp.int32)])
    out = _make_edge_map(da, db, mode, ch, epw)(ta, tb, ia, ib)
    return out[:e]


def _rowsum_sigmoid_body(x_ref, o_ref):
    o_ref[...] = jax.nn.sigmoid(jnp.sum(x_ref[...], axis=1))


@jax.jit
def _rowsum_sigmoid(x):
    morig, k = x.shape
    blk = 8192
    pad = (-morig) % blk
    if pad:
        x = jnp.concatenate([x, jnp.zeros((pad, k), x.dtype)], axis=0)
    m = x.shape[0]
    out = _rowsum_sigmoid_call(x, blk)
    return out[:morig]


@functools.partial(jax.jit, static_argnames=("blk",))
def _rowsum_sigmoid_call(x, blk):
    m, k = x.shape
    return pl.pallas_call(
        _rowsum_sigmoid_body,
        grid=(m // blk,),
        in_specs=[pl.BlockSpec((blk, k), lambda i: (i, 0))],
        out_specs=pl.BlockSpec((blk,), lambda i: (i,)),
        out_shape=jax.ShapeDtypeStruct((m,), jnp.float32),
    )(x)


NPW2 = 640


def _sc_aggregate2(table_a, table_b, ga_pad, gb_pad, starts_pad):
    d1, d2 = table_a.shape[1], table_b.shape[1]
    ch = 512 if d1 + d2 <= 64 else 256
    out = _make_agg2(d1, d2, ch, NPW2, 2 * N_NODES)(
        table_a, table_b, ga_pad, gb_pad, starts_pad)
    return out[:2 * N_NODES]


def _mm_body(x_ref, w_ref, b_ref, o_ref, *, act):
    x = x_ref[...]
    w = w_ref[...]
    acc = jax.lax.dot_general(x, w, (((1,), (0,)), ((), ())),
                              preferred_element_type=jnp.float32)
    acc = acc + b_ref[...][None, :]
    if act == "relu":
        acc = jnp.maximum(acc, 0.0)
    elif act == "sigmoid":
        acc = jax.nn.sigmoid(acc)
    o_ref[...] = acc


@functools.partial(jax.jit, static_argnames=("act", "blk"))
def _mm(x, w, b, act="none", blk=1000):
    m, k = x.shape
    n = w.shape[1]
    pad = (-m) % blk
    if pad:
        x = jnp.concatenate([x, jnp.zeros((pad, k), x.dtype)], axis=0)
    mp = x.shape[0]
    out = pl.pallas_call(
        functools.partial(_mm_body, act=act),
        grid=(mp // blk,),
        in_specs=[
            pl.BlockSpec((blk, k), lambda i: (i, 0)),
            pl.BlockSpec((k, n), lambda i: (0, 0)),
            pl.BlockSpec((n,), lambda i: (0,)),
        ],
        out_specs=pl.BlockSpec((blk, n), lambda i: (i, 0)),
        out_shape=jax.ShapeDtypeStruct((mp, n), jnp.float32),
    )(x, w, b)
    return out[:m] if pad else out


def _mm2_body(x1_ref, w1_ref, x2_ref, w2_ref, b_ref, o_ref):
    acc = jax.lax.dot_general(x1_ref[...], w1_ref[...], (((1,), (0,)), ((), ())),
                              preferred_element_type=jnp.float32)
    acc += jax.lax.dot_general(x2_ref[...], w2_ref[...], (((1,), (0,)), ((), ())),
                               preferred_element_type=jnp.float32)
    o_ref[...] = acc + b_ref[...][None, :]


@functools.partial(jax.jit, static_argnames=("blk",))
def _mm2(x1, w1, x2, w2, b, blk=1000):
    m, k1 = x1.shape
    k2 = x2.shape[1]
    n = w1.shape[1]
    pad = (-m) % blk
    if pad:
        x1 = jnp.concatenate([x1, jnp.zeros((pad, k1), x1.dtype)], axis=0)
        x2 = jnp.concatenate([x2, jnp.zeros((pad, k2), x2.dtype)], axis=0)
    mp = x1.shape[0]
    out = pl.pallas_call(
        _mm2_body,
        grid=(mp // blk,),
        in_specs=[
            pl.BlockSpec((blk, k1), lambda i: (i, 0)),
            pl.BlockSpec((k1, n), lambda i: (0, 0)),
            pl.BlockSpec((blk, k2), lambda i: (i, 0)),
            pl.BlockSpec((k2, n), lambda i: (0, 0)),
            pl.BlockSpec((n,), lambda i: (0,)),
        ],
        out_specs=pl.BlockSpec((blk, n), lambda i: (i, 0)),
        out_shape=jax.ShapeDtypeStruct((mp, n), jnp.float32),
    )(x1, w1, x2, w2, b)
    return out[:m] if pad else out


def _mm_pair_body(x_ref, w_ref, b_ref, o_ref, *, act):
    acc = jax.lax.dot_general(x_ref[...], w_ref[0], (((1,), (0,)), ((), ())),
                              preferred_element_type=jnp.float32)
    acc = acc + b_ref[0]
    if act == "relu":
        acc = jnp.maximum(acc, 0.0)
    o_ref[...] = acc


@functools.partial(jax.jit, static_argnames=("act", "blk"))
def _mm_pair(x, w2, b2, act="none", blk=1000):
    m2, k = x.shape
    m = m2 // 2
    n = w2.shape[2]
    hb = m // blk
    out = pl.pallas_call(
        functools.partial(_mm_pair_body, act=act),
        grid=(2, hb),
        in_specs=[
            pl.BlockSpec((blk, k), lambda h, i: (h * hb + i, 0)),
            pl.BlockSpec((1, k, n), lambda h, i: (h, 0, 0)),
            pl.BlockSpec((1, 1, n), lambda h, i: (h, 0, 0)),
        ],
        out_specs=pl.BlockSpec((blk, n), lambda h, i: (h * hb + i, 0)),
        out_shape=jax.ShapeDtypeStruct((m2, n), jnp.float32),
    )(x, w2, b2[:, None, :])
    return out


def _mm_pair2_body(x1_ref, w1_ref, x2_ref, w2_ref, b_ref, o_ref):
    acc = jax.lax.dot_general(x1_ref[...], w1_ref[0], (((1,), (0,)), ((), ())),
                              preferred_element_type=jnp.float32)
    acc += jax.lax.dot_general(x2_ref[...], w2_ref[0], (((1,), (0,)), ((), ())),
                               preferred_element_type=jnp.float32)
    o_ref[...] = acc + b_ref[0]


@functools.partial(jax.jit, static_argnames=("blk",))
def _mm_pair2(x1, w1s, x2, w2s, b2, blk=1000):
    m2, k1 = x1.shape
    m = m2 // 2
    k2 = x2.shape[1]
    n = w1s.shape[2]
    hb = m // blk
    out = pl.pallas_call(
        _mm_pair2_body,
        grid=(2, hb),
        in_specs=[
            pl.BlockSpec((blk, k1), lambda h, i: (h * hb + i, 0)),
            pl.BlockSpec((1, k1, n), lambda h, i: (h, 0, 0)),
            pl.BlockSpec((blk, k2), lambda h, i: (h * hb + i, 0)),
            pl.BlockSpec((1, k2, n), lambda h, i: (h, 0, 0)),
            pl.BlockSpec((1, 1, n), lambda h, i: (h, 0, 0)),
        ],
        out_specs=pl.BlockSpec((blk, n), lambda h, i: (h * hb + i, 0)),
        out_shape=jax.ShapeDtypeStruct((m2, n), jnp.float32),
    )(x1, w1s, x2, w2s, b2[:, None, :])
    return out


def _mlp(h, layers, final_act="none"):
    last = len(layers) - 1
    for i, p in enumerate(layers):
        h = _mm(h, p["W"], p["b"], act=("relu" if i != last else final_act))
    return h


def _mlp_pair(h, layers_u, layers_v):
    last = len(layers_u) - 1
    for i, (pu, pv) in enumerate(zip(layers_u, layers_v)):
        w2 = jnp.stack([pu["W"], pv["W"]])
        b2 = jnp.stack([pu["b"], pv["b"]])
        h = _mm_pair(h, w2, b2, act=("relu" if i != last else "none"))
    return h


def _bean_conv(xuv, xe, srt, p, self_loop, relu_out):
    ga_pad, gb_pad, starts_pad, row_p, colN_p = srt
    agg = _sc_aggregate2(xuv, xe, ga_pad, gb_pad, starts_pad)
    wu, wv = p["u"]["W"], p["v"]["W"]
    bs = jnp.stack([p["u"]["b"], p["v"]["b"]])
    if self_loop:
        d = xuv.shape[1]
        ws_top = jnp.stack([wu[:d], wv[:d]])
        ws_bot = jnp.stack([wu[d:], wv[d:]])
        xuv_n = _mm_pair2(xuv, ws_top, agg, ws_bot, bs)
    else:
        xuv_n = _mm_pair(agg, jnp.stack([wu, wv]), bs)
    if "e" in p:
        ou = wu.shape[1]
        we = p["e"]["W"]
        wes = jnp.stack([we[:ou], we[ou:]])
        bes = jnp.stack([p["e"]["b"], jnp.zeros_like(p["e"]["b"])])
        ppair = _mm_pair(xuv_n, wes, bes)
        xe_n = _sc_edge_map(ppair, ppair, row_p, colN_p,
                            "add_relu" if relu_out else "add")
    else:
        xe_n = _sc_edge_map(xuv_n, xuv_n, row_p, colN_p, "concat")
    if relu_out:
        xuv_n = jnp.maximum(xuv_n, 0.0)
    return xuv_n, xe_n


def _pad_idx(ix, ch=512):
    e = ix.shape[0]
    epw = -(-e // (NW * ch)) * ch
    return jnp.concatenate([ix, jnp.zeros((NW * epw - e,), jnp.int32)])


def kernel(xu, xv, xe, params, adj_row, adj_col, ep_row, ep_col):
    n = N_NODES
    perm_r = jnp.argsort(adj_row).astype(jnp.int32)
    perm_c = jnp.argsort(adj_col).astype(jnp.int32)
    row_s = adj_row[perm_r]
    col_s = adj_col[perm_c]
    grid = jnp.arange(n + 1, dtype=jnp.int32)
    starts_r = jnp.searchsorted(row_s, grid).astype(jnp.int32)
    starts_c = jnp.searchsorted(col_s, grid).astype(jnp.int32)
    starts_comb = jnp.concatenate([starts_r[:n], N_EDGES + starts_c])
    spad = NW * NPW2 + 8 - starts_comb.shape[0]
    starts_pad = jnp.concatenate(
        [starts_comb, jnp.full((spad,), 2 * N_EDGES, jnp.int32)])
    ga = jnp.concatenate([n + adj_col[perm_r], adj_row[perm_c]])
    gb = jnp.concatenate([perm_r, perm_c])
    zpad = jnp.zeros((1024,), jnp.int32)
    ga_pad = jnp.concatenate([ga, zpad])
    gb_pad = jnp.concatenate([gb, zpad])
    row_p = _pad_idx(adj_row)
    colN_p = _pad_idx(n + adj_col)
    srt = (ga_pad, gb_pad, starts_pad, row_p, colN_p)

    xuv = jnp.concatenate([xu, xv])
    n_enc = len(params["enc"])
    for i, p in enumerate(params["enc"]):
        xuv, xe = _bean_conv(xuv, xe, srt, p, i != n_enc - 1, i != n_enc - 1)
    zuv = xuv
    n_dec = len(params["dec"])
    for i, p in enumerate(params["dec"]):
        xuv, xe = _bean_conv(xuv, xe, srt, p, True, i != n_dec - 1)
    nprob_uv = _mlp(zuv, params["clf"], final_act="sigmoid")
    zuv2 = _mlp_pair(zuv, params["umlp"], params["vmlp"])
    g = _sc_edge_map(zuv2, zuv2, _pad_idx(ep_row), _pad_idx(n + ep_col), "mul")
    eprob = _rowsum_sigmoid(g)[:ep_row.shape[0]]
    return (xuv[:n], xuv[n:], xe[:N_EDGES], zuv[:n], zuv[n:], eprob,
            nprob_uv[:n], nprob_uv[n:])
